# Initial kernel scaffold; baseline (speedup 1.0000x reference)
import jax
import jax.numpy as jnp
from jax.experimental import pallas as pl


def kernel(x, router_w, router_b, w1, b1, w2, b2, width_emb):
    raise NotImplementedError("write your pallas kernel here")


# speedup vs baseline: 2.5476x; 2.5476x over previous
import functools

import jax
import jax.numpy as jnp
from jax import lax
from jax.experimental import pallas as pl
from jax.experimental.pallas import tpu as pltpu

_B, _S, _D = 1, 2048, 768
_E, _K, _FFN = 8, 2, 1536
_TB = 256


def _router_body(x_ref, rw_ref, rb_ref, comb_ref):
    xb = x_ref[...]
    logits = lax.dot_general(
        xb, rw_ref[...], (((1,), (1,)), ((), ())),
        preferred_element_type=jnp.float32)
    logits = logits + rb_ref[...]
    m = jnp.max(logits, axis=-1, keepdims=True)
    p = jnp.exp(logits - m)
    p = p / jnp.sum(p, axis=-1, keepdims=True)
    ii = lax.broadcasted_iota(jnp.int32, (_TB, _E), 1)
    v1 = jnp.max(p, axis=-1, keepdims=True)
    i1 = jnp.min(jnp.where(p == v1, ii, _E), axis=-1, keepdims=True)
    m1 = ii == i1
    p2 = jnp.where(m1, -1.0, p)
    v2 = jnp.max(p2, axis=-1, keepdims=True)
    i2 = jnp.min(jnp.where(p2 == v2, ii, _E), axis=-1, keepdims=True)
    m2 = ii == i2
    s = v1 + v2 + 1e-9
    comb_ref[...] = (v1 / s) * m1.astype(jnp.float32) + \
                    (v2 / s) * m2.astype(jnp.float32)


def _moe_body(x_ref, comb_ref, w1_ref, b1_ref, w2_ref, b2_ref, we_ref,
              out_ref):
    t = pl.program_id(0)
    e = pl.program_id(1)
    h = x_ref[...] + we_ref[0]
    y = lax.dot_general(
        h, w1_ref[0], (((1,), (1,)), ((), ())),
        preferred_element_type=jnp.float32)
    y = y + b1_ref[0]
    y = y * 0.5 * (1.0 + lax.erf(y * (2.0 ** -0.5)))
    z = lax.dot_general(
        y, w2_ref[0], (((1,), (1,)), ((), ())),
        preferred_element_type=jnp.float32)
    r = h + z + b2_ref[0]
    mu = jnp.mean(r, axis=-1, keepdims=True)
    var = jnp.mean((r - mu) ** 2, axis=-1, keepdims=True)
    eo = (r - mu) * lax.rsqrt(var + 1e-5)
    cb = comb_ref[pl.ds(t * _TB, _TB), :]
    ii = lax.broadcasted_iota(jnp.int32, (_TB, _E), 1)
    c = jnp.sum(jnp.where(ii == e, cb, 0.0), axis=-1, keepdims=True)
    acc = c * eo

    @pl.when(e == 0)
    def _():
        out_ref[...] = acc

    @pl.when(e > 0)
    def _():
        out_ref[...] = out_ref[...] + acc


def kernel(x, router_w, router_b, w1, b1, w2, b2, width_emb):
    T = x.shape[0] * x.shape[1]
    flat = x.reshape(T, _D)
    combine = pl.pallas_call(
        _router_body,
        grid=(T // _TB,),
        in_specs=[
            pl.BlockSpec((_TB, _D), lambda t: (t, 0)),
            pl.BlockSpec((_E, _D), lambda t: (0, 0)),
            pl.BlockSpec((1, _E), lambda t: (0, 0)),
        ],
        out_specs=pl.BlockSpec((_TB, _E), lambda t: (t, 0)),
        out_shape=jax.ShapeDtypeStruct((T, _E), jnp.float32),
    )(flat, router_w, router_b.reshape(1, _E))

    out = pl.pallas_call(
        _moe_body,
        grid=(T // _TB, _E),
        in_specs=[
            pl.BlockSpec((_TB, _D), lambda t, e: (t, 0)),
            pl.BlockSpec((T, _E), lambda t, e: (0, 0)),
            pl.BlockSpec((1, _FFN, _D), lambda t, e: (e, 0, 0)),
            pl.BlockSpec((1, 1, _FFN), lambda t, e: (e, 0, 0)),
            pl.BlockSpec((1, _D, _FFN), lambda t, e: (e, 0, 0)),
            pl.BlockSpec((1, 1, _D), lambda t, e: (e, 0, 0)),
            pl.BlockSpec((1, 1, _D), lambda t, e: (e, 0, 0)),
        ],
        out_specs=pl.BlockSpec((_TB, _D), lambda t, e: (t, 0)),
        out_shape=jax.ShapeDtypeStruct((T, _D), jnp.float32),
    )(flat, combine, w1, b1.reshape(_E, 1, _FFN), w2,
      b2.reshape(_E, 1, _D), width_emb.reshape(_E, 1, _D))
    return out.reshape(x.shape)
